# baseline (speedup 1.0000x reference)
import jax
import jax.numpy as jnp
import numpy as np
from jax.experimental import pallas as pl

_FACTOR = 8

_rng = np.random.RandomState(0)
_FIXED_IDX = np.stack(
    [_rng.choice(_FACTOR, _FACTOR, replace=False),
     _rng.choice(_FACTOR, _FACTOR, replace=False)],
).astype(np.int32)

_FIXED_PAD = np.zeros((8, 128), np.int32)
_FIXED_PAD[:2, :_FACTOR] = _FIXED_IDX

_C_BLOCK = 8


def _body(idx_ref, img_ref, fixed_ref, out_img_ref, out_idx_ref):
    out_img_ref[...] = img_ref[...]
    idx = idx_ref[...]
    nz = jnp.any(idx != 0)
    out_idx_ref[...] = jnp.where(nz, idx, fixed_ref[...])


def kernel(img, indices):
    c, h, w = img.shape
    idx_pad = jnp.zeros((8, 128), jnp.int32).at[:2, :_FACTOR].set(indices)
    fixed_pad = jnp.asarray(_FIXED_PAD)

    grid = (c // _C_BLOCK,)
    out_img, out_idx_pad = pl.pallas_call(
        _body,
        grid=grid,
        in_specs=[
            pl.BlockSpec((8, 128), lambda i: (0, 0)),
            pl.BlockSpec((_C_BLOCK, h, w), lambda i: (i, 0, 0)),
            pl.BlockSpec((8, 128), lambda i: (0, 0)),
        ],
        out_specs=[
            pl.BlockSpec((_C_BLOCK, h, w), lambda i: (i, 0, 0)),
            pl.BlockSpec((8, 128), lambda i: (0, 0)),
        ],
        out_shape=[
            jax.ShapeDtypeStruct((c, h, w), img.dtype),
            jax.ShapeDtypeStruct((8, 128), jnp.int32),
        ],
    )(idx_pad, img, fixed_pad)

    return out_img, out_idx_pad[:2, :_FACTOR]
